# baseline (speedup 1.0000x reference)
import functools

import jax
import jax.numpy as jnp
from jax import lax
from jax.experimental import pallas as pl
from jax.experimental.pallas import tpu as pltpu
from jax.experimental.pallas import tpu_sc as plsc

_CHUNK = 128


@functools.cache
def _sc_geometry():
    info = plsc.get_sparse_core_info()
    return info.num_cores, info.num_subcores


@functools.partial(jax.jit, static_argnums=(2, 3, 4))
def _gather_rows(table, idx2d, b_per_w, n_chunks, nc):
    V, D = table.shape
    B = idx2d.shape[0] * idx2d.shape[1]
    mesh = plsc.VectorSubcoreMesh(core_axis_name="c", subcore_axis_name="s")

    @functools.partial(
        pl.kernel,
        mesh=mesh,
        out_type=jax.ShapeDtypeStruct((B, D), jnp.float32),
        scratch_types=[
            pltpu.VMEM((n_chunks, _CHUNK), jnp.int32),
            pltpu.VMEM((b_per_w, D), jnp.float32),
            pltpu.SemaphoreType.DMA,
        ],
        compiler_params=pltpu.CompilerParams(use_tc_tiling_on_sc=False),
    )
    def k(table_hbm, idx_hbm, out_hbm, idx_v, rows_v, sem):
        wid = lax.axis_index("s") * nc + lax.axis_index("c")
        base = wid * b_per_w
        pltpu.sync_copy(idx_hbm.at[pl.ds(wid * n_chunks, n_chunks)], idx_v)
        copies = []
        for j in range(n_chunks):
            copies.append(
                pltpu.async_copy(
                    table_hbm.at[idx_v.at[j]],
                    rows_v.at[pl.ds(j * _CHUNK, _CHUNK)],
                    sem,
                )
            )
        for c in copies:
            c.wait()
        pltpu.sync_copy(rows_v, out_hbm.at[pl.ds(base, b_per_w)])

    return k(table, idx2d)


def kernel(boxes, box_indices):
    nm, nb, two, dims = boxes.shape
    D = two * dims
    B = box_indices.shape[0]
    nc, ns = _sc_geometry()
    nw = nc * ns
    table = boxes.reshape(nb, D)
    b_per_w = B // nw
    n_chunks = b_per_w // _CHUNK
    idx2d = box_indices.astype(jnp.int32).reshape(nw * n_chunks, _CHUNK)
    out = _gather_rows(table, idx2d, b_per_w, n_chunks, nc)
    return out.reshape(nm, B, two, dims)
